# baseline (speedup 1.0000x reference)
import functools

import jax
import jax.numpy as jnp
from jax import lax
from jax.experimental import pallas as pl
from jax.experimental.pallas import tpu as pltpu
from jax.experimental.pallas import tpu_sc as plsc

D = 64
B = 4096
L = 50
NC = 2
NS = 16
NW = NC * NS
S_PER_W = B // NW
NBUF = 8
NGROUP = S_PER_W // NBUF


def _gather_body(idx_hbm, table_hbm, out_hbm, idx_v, rows_v, sems):
    wid = lax.axis_index("s") * NC + lax.axis_index("c")
    base = wid * S_PER_W
    pltpu.sync_copy(idx_hbm.at[pl.ds(base, S_PER_W)], idx_v)

    def start_gather(s, b):
        pltpu.async_copy(table_hbm.at[idx_v.at[s]], rows_v.at[b], sems.at[b])

    for b in range(NBUF):
        start_gather(b, b)

    def group(g, carry):
        for b in range(NBUF):
            s = g * NBUF + b
            pltpu.make_async_copy(
                table_hbm.at[idx_v.at[s]], rows_v.at[b], sems.at[b]
            ).wait()
            pltpu.sync_copy(rows_v.at[b], out_hbm.at[base + s])

            @pl.when(s + NBUF < S_PER_W)
            def _():
                start_gather(s + NBUF, b)
        return carry

    lax.fori_loop(0, NGROUP, group, 0)


@jax.jit
def _embed_lookup(sentences, embed_table):
    mesh = plsc.VectorSubcoreMesh(core_axis_name="c", subcore_axis_name="s")
    fn = functools.partial(
        pl.kernel,
        mesh=mesh,
        out_type=jax.ShapeDtypeStruct((B, L, D), jnp.float32),
        scratch_types=[
            pltpu.VMEM((S_PER_W, L), jnp.int32),
            pltpu.VMEM((NBUF, L, D), jnp.float32),
            pltpu.SemaphoreType.DMA((NBUF,)),
        ],
        compiler_params=pltpu.CompilerParams(use_tc_tiling_on_sc=False),
    )(_gather_body)
    return fn(sentences, embed_table)


def kernel(sentences, embed_table):
    if sentences.dtype != jnp.int32:
        sentences = sentences.astype(jnp.int32)
    return _embed_lookup(sentences, embed_table)


# speedup vs baseline: 1.0628x; 1.0628x over previous
import functools

import jax
import jax.numpy as jnp
from jax import lax
from jax.experimental import pallas as pl
from jax.experimental.pallas import tpu as pltpu
from jax.experimental.pallas import tpu_sc as plsc

D = 64
B = 4096
L = 50
NC = 2
NS = 16
NW = NC * NS
S_PER_W = B // NW
NBUF = 8
NGROUP = S_PER_W // NBUF


DP = 128


def _gather_body(idx_hbm, table_hbm, out_hbm, idx_v, rows_v, sems):
    wid = lax.axis_index("s") * NC + lax.axis_index("c")
    base = wid * S_PER_W
    pltpu.sync_copy(idx_hbm.at[pl.ds(base, S_PER_W)], idx_v)

    def start_gather(s, b):
        pltpu.async_copy(table_hbm.at[idx_v.at[s]], rows_v.at[b], sems.at[b])

    for b in range(NBUF):
        start_gather(b, b)

    def group(g, carry):
        for b in range(NBUF):
            s = g * NBUF + b
            pltpu.make_async_copy(
                table_hbm.at[idx_v.at[s]], rows_v.at[b], sems.at[b]
            ).wait()
            pltpu.sync_copy(rows_v.at[b, :, pl.ds(0, D)], out_hbm.at[base + s])

            @pl.when(s + NBUF < S_PER_W)
            def _():
                start_gather(s + NBUF, b)
        return carry

    lax.fori_loop(0, NGROUP, group, 0)


@jax.jit
def _embed_lookup(sentences, embed_table_padded):
    mesh = plsc.VectorSubcoreMesh(core_axis_name="c", subcore_axis_name="s")
    fn = functools.partial(
        pl.kernel,
        mesh=mesh,
        out_type=jax.ShapeDtypeStruct((B, L, D), jnp.float32),
        scratch_types=[
            pltpu.VMEM((S_PER_W, L), jnp.int32),
            pltpu.VMEM((NBUF, L, DP), jnp.float32),
            pltpu.SemaphoreType.DMA((NBUF,)),
        ],
        compiler_params=pltpu.CompilerParams(use_tc_tiling_on_sc=False),
    )(_gather_body)
    return fn(sentences, embed_table_padded)


def kernel(sentences, embed_table):
    if sentences.dtype != jnp.int32:
        sentences = sentences.astype(jnp.int32)
    table_padded = jnp.pad(embed_table, ((0, 0), (0, DP - D)))
    return _embed_lookup(sentences, table_padded)


# speedup vs baseline: 1.4666x; 1.3800x over previous
import functools

import jax
import jax.numpy as jnp
from jax import lax
from jax.experimental import pallas as pl
from jax.experimental.pallas import tpu as pltpu
from jax.experimental.pallas import tpu_sc as plsc

D = 64
B = 4096
L = 50
TOTAL = B * L
NC = 2
NS = 16
NW = NC * NS
S_PER_W = B // NW
NBUF = 4
NGROUP = S_PER_W // NBUF

_LOADS = [(0, range(0, 16)), (16, range(0, 16)), (32, range(0, 16)),
          (34, range(14, 16))]


def _gather_body(idx_hbm, table_hbm, out_hbm, idx_v, rows_v, gsems):
    wid = lax.axis_index("s") * NC + lax.axis_index("c")
    base = wid * S_PER_W
    pltpu.sync_copy(idx_hbm.at[pl.ds(base * L, S_PER_W * L)], idx_v)

    def issue(s, b):
        w = 0
        for off, lanes in _LOADS:
            vals = idx_v[pl.ds(s * L + off, 16)]
            for j in lanes:
                pltpu.async_copy(
                    table_hbm.at[pl.ds(vals[j], 1)],
                    rows_v.at[b, pl.ds(w, 1)],
                    gsems.at[b],
                )
                w += 1

    def drain(b):
        for w in range(L):
            pltpu.make_async_copy(
                table_hbm.at[pl.ds(0, 1)], rows_v.at[b, pl.ds(w, 1)], gsems.at[b]
            ).wait()

    for b in range(NBUF):
        issue(b, b)

    def group(g, carry):
        for b in range(NBUF):
            s = g * NBUF + b
            drain(b)
            pltpu.sync_copy(rows_v.at[b], out_hbm.at[base + s])

            @pl.when(s + NBUF < S_PER_W)
            def _():
                issue(s + NBUF, b)
        return carry

    lax.fori_loop(0, NGROUP, group, 0)


@jax.jit
def _embed_lookup(idx_flat, embed_table):
    mesh = plsc.VectorSubcoreMesh(core_axis_name="c", subcore_axis_name="s")
    fn = functools.partial(
        pl.kernel,
        mesh=mesh,
        out_type=jax.ShapeDtypeStruct((B, L, D), jnp.float32),
        scratch_types=[
            pltpu.VMEM((S_PER_W * L,), jnp.int32),
            pltpu.VMEM((NBUF, L, D), jnp.float32),
            pltpu.SemaphoreType.DMA((NBUF,)),
        ],
        compiler_params=pltpu.CompilerParams(use_tc_tiling_on_sc=True),
    )(_gather_body)
    return fn(idx_flat, embed_table)


def kernel(sentences, embed_table):
    idx_flat = sentences.reshape(TOTAL).astype(jnp.int32)
    return _embed_lookup(idx_flat, embed_table)


# speedup vs baseline: 1.5149x; 1.0329x over previous
import functools

import jax
import jax.numpy as jnp
from jax import lax
from jax.experimental import pallas as pl
from jax.experimental.pallas import tpu as pltpu
from jax.experimental.pallas import tpu_sc as plsc

D = 64
B = 4096
L = 50
TOTAL = B * L
NC = 2
NS = 16
NW = NC * NS
S_PER_W = B // NW
NBUF = 4
NGROUP = S_PER_W // NBUF

_LOADS = [(0, range(0, 16)), (16, range(0, 16)), (32, range(0, 16)),
          (34, range(14, 16))]


def _gather_body(idx_hbm, table_hbm, out_hbm, idx_v, rows_v, gsems):
    wid = lax.axis_index("s") * NC + lax.axis_index("c")
    base = wid * S_PER_W
    pltpu.sync_copy(idx_hbm.at[pl.ds(base * L, S_PER_W * L)], idx_v)

    def issue(s, b):
        w = 0
        for off, lanes in _LOADS:
            vals = idx_v[pl.ds(s * L + off, 16)]
            for j in lanes:
                pltpu.async_copy(
                    table_hbm.at[pl.ds(vals[j], 1)],
                    rows_v.at[b, pl.ds(w, 1)],
                    gsems.at[b],
                )
                w += 1

    def drain(b):
        pltpu.make_async_copy(
            out_hbm.at[base], rows_v.at[b], gsems.at[b]
        ).wait()

    for b in range(NBUF):
        issue(b, b)

    def group(g, carry):
        for b in range(NBUF):
            s = g * NBUF + b
            drain(b)
            pltpu.sync_copy(rows_v.at[b], out_hbm.at[base + s])

            @pl.when(s + NBUF < S_PER_W)
            def _():
                issue(s + NBUF, b)
        return carry

    lax.fori_loop(0, NGROUP, group, 0)


@jax.jit
def _embed_lookup(idx_flat, embed_table):
    mesh = plsc.VectorSubcoreMesh(core_axis_name="c", subcore_axis_name="s")
    fn = functools.partial(
        pl.kernel,
        mesh=mesh,
        out_type=jax.ShapeDtypeStruct((B, L, D), jnp.float32),
        scratch_types=[
            pltpu.VMEM((S_PER_W * L,), jnp.int32),
            pltpu.VMEM((NBUF, L, D), jnp.float32),
            pltpu.SemaphoreType.DMA((NBUF,)),
        ],
        compiler_params=pltpu.CompilerParams(use_tc_tiling_on_sc=True),
    )(_gather_body)
    return fn(idx_flat, embed_table)


def kernel(sentences, embed_table):
    idx_flat = sentences.reshape(TOTAL).astype(jnp.int32)
    return _embed_lookup(idx_flat, embed_table)
